# Initial kernel scaffold; baseline (speedup 1.0000x reference)
import jax
import jax.numpy as jnp
from jax import lax
from jax.experimental import pallas as pl
from jax.experimental.pallas import tpu as pltpu
from jax.experimental.pallas import tpu_sc as plsc

B, L = 4096, 200
D = 64
N_TOK = B * L
NC, NS, LANES = 2, 16, 16
NW = NC * NS
TOK_PER_W = N_TOK // NW
CHUNK = 256
N_CHUNKS = TOK_PER_W // CHUNK
IDX_MINOR = 128
N_IDX_ROWS = CHUNK // IDX_MINOR


def _sc_body(traj_ref, t_ref, l_ref, u_ref, out_ref,
             raw_v, tix_v, lix_v, uix_v, tbuf_v, lbuf_v, ubuf_v, zbuf_v, sem):
    wid = lax.axis_index("s") * NC + lax.axis_index("c")
    w_base = wid * TOK_PER_W

    def zero_row(i, carry):
        for c in range(2 * D // LANES):
            zbuf_v[i, pl.ds(c * LANES, LANES)] = jnp.zeros((LANES,), jnp.float32)
        return carry
    lax.fori_loop(0, CHUNK, zero_row, 0)

    lane = lax.iota(jnp.int32, 16)

    def chunk_body(i, carry):
        base = w_base + i * CHUNK
        pltpu.sync_copy(traj_ref.at[pl.ds(base * 3, CHUNK * 3)], raw_v)

        for g in range(CHUNK // LANES):
            triple = lane * 3 + (g * LANES * 3)
            u_i = plsc.load_gather(raw_v, [triple])
            l_i = plsc.load_gather(raw_v, [triple + 1])
            t_i = plsc.load_gather(raw_v, [triple + 2])
            t_i = lax.rem(t_i + 23, 24) + 1
            j, c = (g * LANES) // IDX_MINOR, (g * LANES) % IDX_MINOR
            uix_v[j, pl.ds(c, LANES)] = u_i
            lix_v[j, pl.ds(c, LANES)] = l_i
            tix_v[j, pl.ds(c, LANES)] = t_i

        handles = []
        for j in range(N_IDX_ROWS):
            rows = pl.ds(j * IDX_MINOR, IDX_MINOR)
            handles.append(pltpu.async_copy(t_ref.at[tix_v.at[j]], tbuf_v.at[rows], sem))
            handles.append(pltpu.async_copy(l_ref.at[lix_v.at[j]], lbuf_v.at[rows], sem))
            handles.append(pltpu.async_copy(u_ref.at[uix_v.at[j]], ubuf_v.at[rows], sem))
        for h in handles:
            h.wait()

        out_rows = pl.ds(base, CHUNK)
        pltpu.sync_copy(tbuf_v, out_ref.at[out_rows, pl.ds(0 * D, D)])
        pltpu.sync_copy(lbuf_v, out_ref.at[out_rows, pl.ds(1 * D, D)])
        pltpu.sync_copy(ubuf_v, out_ref.at[out_rows, pl.ds(2 * D, D)])
        pltpu.sync_copy(zbuf_v, out_ref.at[out_rows, pl.ds(3 * D, 2 * D)])
        return carry

    lax.fori_loop(0, N_CHUNKS, chunk_body, 0)


def _multi_embed(traj_flat, embed_t_w, embed_l_w, embed_u_w):
    fn = pl.kernel(
        _sc_body,
        out_type=jax.ShapeDtypeStruct((N_TOK, 5 * D), jnp.float32),
        mesh=plsc.VectorSubcoreMesh(core_axis_name="c", subcore_axis_name="s"),
        scratch_types=[
            pltpu.VMEM((CHUNK * 3,), jnp.int32),
            pltpu.VMEM((N_IDX_ROWS, IDX_MINOR), jnp.int32),
            pltpu.VMEM((N_IDX_ROWS, IDX_MINOR), jnp.int32),
            pltpu.VMEM((N_IDX_ROWS, IDX_MINOR), jnp.int32),
            pltpu.VMEM((CHUNK, D), jnp.float32),
            pltpu.VMEM((CHUNK, D), jnp.float32),
            pltpu.VMEM((CHUNK, D), jnp.float32),
            pltpu.VMEM((CHUNK, 2 * D), jnp.float32),
            pltpu.SemaphoreType.DMA,
        ],
    )
    return fn(traj_flat, embed_t_w, embed_l_w, embed_u_w)


def kernel(trajectories, embed_t_w, embed_l_w, embed_u_w):
    traj_flat = trajectories.reshape(-1)
    out = _multi_embed(traj_flat, embed_t_w, embed_l_w, embed_u_w)
    return out.reshape(B, L, 5 * D)


# speedup vs baseline: 1.2541x; 1.2541x over previous
import jax
import jax.numpy as jnp
from jax import lax
from jax.experimental import pallas as pl
from jax.experimental.pallas import tpu as pltpu
from jax.experimental.pallas import tpu_sc as plsc

B, L = 4096, 200
D = 64
N_TOK = B * L
NC, NS, LANES = 2, 16, 16
NW = NC * NS
TOK_PER_W = N_TOK // NW
CHUNK = 256
N_CHUNKS = TOK_PER_W // CHUNK
IDX_MINOR = 128
N_IDX_ROWS = CHUNK // IDX_MINOR


def _sc_body(uix_hbm, lix_hbm, traw_hbm, t_ref, l_ref, u_ref, out_ref,
             uix_v, lix_v, tix_v, tbuf_v, lbuf_v, ubuf_v, zbuf_v, sem):
    wid = lax.axis_index("s") * NC + lax.axis_index("c")
    w_base = wid * TOK_PER_W

    def zero_row(i, carry):
        for c in range(2 * D // LANES):
            zbuf_v[i, pl.ds(c * LANES, LANES)] = jnp.zeros((LANES,), jnp.float32)
        return carry
    lax.fori_loop(0, CHUNK, zero_row, 0)

    def chunk_body(i, carry):
        base = w_base + i * CHUNK
        for j in range(N_IDX_ROWS):
            seg = pl.ds(base + j * IDX_MINOR, IDX_MINOR)
            pltpu.sync_copy(uix_hbm.at[seg], uix_v.at[j])
            pltpu.sync_copy(lix_hbm.at[seg], lix_v.at[j])
            pltpu.sync_copy(traw_hbm.at[seg], tix_v.at[j])

        for j in range(N_IDX_ROWS):
            for c in range(IDX_MINOR // LANES):
                t_i = tix_v[j, pl.ds(c * LANES, LANES)]
                tix_v[j, pl.ds(c * LANES, LANES)] = lax.rem(t_i + 23, 24) + 1

        handles = []
        for j in range(N_IDX_ROWS):
            rows = pl.ds(j * IDX_MINOR, IDX_MINOR)
            handles.append(pltpu.async_copy(t_ref.at[tix_v.at[j]], tbuf_v.at[rows], sem))
            handles.append(pltpu.async_copy(l_ref.at[lix_v.at[j]], lbuf_v.at[rows], sem))
            handles.append(pltpu.async_copy(u_ref.at[uix_v.at[j]], ubuf_v.at[rows], sem))
        for h in handles:
            h.wait()

        out_rows = pl.ds(base, CHUNK)
        pltpu.sync_copy(tbuf_v, out_ref.at[out_rows, pl.ds(0 * D, D)])
        pltpu.sync_copy(lbuf_v, out_ref.at[out_rows, pl.ds(1 * D, D)])
        pltpu.sync_copy(ubuf_v, out_ref.at[out_rows, pl.ds(2 * D, D)])
        pltpu.sync_copy(zbuf_v, out_ref.at[out_rows, pl.ds(3 * D, 2 * D)])
        return carry

    lax.fori_loop(0, N_CHUNKS, chunk_body, 0)


def _multi_embed(u_idx, l_idx, t_raw, embed_t_w, embed_l_w, embed_u_w):
    fn = pl.kernel(
        _sc_body,
        out_type=jax.ShapeDtypeStruct((N_TOK, 5 * D), jnp.float32),
        mesh=plsc.VectorSubcoreMesh(core_axis_name="c", subcore_axis_name="s"),
        compiler_params=pltpu.CompilerParams(use_tc_tiling_on_sc=False),
        scratch_types=[
            pltpu.VMEM((N_IDX_ROWS, IDX_MINOR), jnp.int32),
            pltpu.VMEM((N_IDX_ROWS, IDX_MINOR), jnp.int32),
            pltpu.VMEM((N_IDX_ROWS, IDX_MINOR), jnp.int32),
            pltpu.VMEM((CHUNK, D), jnp.float32),
            pltpu.VMEM((CHUNK, D), jnp.float32),
            pltpu.VMEM((CHUNK, D), jnp.float32),
            pltpu.VMEM((CHUNK, 2 * D), jnp.float32),
            pltpu.SemaphoreType.DMA,
        ],
    )
    return fn(u_idx, l_idx, t_raw, embed_t_w, embed_l_w, embed_u_w)


def kernel(trajectories, embed_t_w, embed_l_w, embed_u_w):
    flat = trajectories.reshape(N_TOK, 3)
    u_idx = flat[:, 0]
    l_idx = flat[:, 1]
    t_raw = flat[:, 2]
    out = _multi_embed(u_idx, l_idx, t_raw, embed_t_w, embed_l_w, embed_u_w)
    return out.reshape(B, L, 5 * D)
